# baseline (speedup 1.0000x reference)
import jax
import jax.numpy as jnp
from jax.experimental import pallas as pl
from jax.experimental.pallas import tpu as pltpu


_NUM_DMAS = 16


def _copy_body(src_ref, dst_ref, sems):
    rows = src_ref.shape[0] // _NUM_DMAS
    copies = [
        pltpu.make_async_copy(
            src_ref.at[pl.ds(i * rows, rows)],
            dst_ref.at[pl.ds(i * rows, rows)],
            sems.at[i],
        )
        for i in range(_NUM_DMAS)
    ]
    for cp in copies:
        cp.start()
    for cp in copies:
        cp.wait()


def kernel(inputs, weight):
    bsz, seq_len = inputs.shape[:2]
    return pl.pallas_call(
        _copy_body,
        out_shape=jax.ShapeDtypeStruct((seq_len, weight.shape[1]), weight.dtype),
        in_specs=[pl.BlockSpec(memory_space=pl.ANY)],
        out_specs=pl.BlockSpec(memory_space=pl.ANY),
        scratch_shapes=[pltpu.SemaphoreType.DMA((_NUM_DMAS,))],
    )(weight)


# speedup vs baseline: 41.2467x; 41.2467x over previous
import jax
import jax.numpy as jnp
from jax.experimental import pallas as pl
from jax.experimental.pallas import tpu as pltpu


_BLOCK_ROWS = 512


def _copy_body(src_ref, dst_ref):
    dst_ref[...] = src_ref[...]


def kernel(inputs, weight):
    bsz, seq_len = inputs.shape[:2]
    dim = weight.shape[1]
    grid = seq_len // _BLOCK_ROWS
    return pl.pallas_call(
        _copy_body,
        out_shape=jax.ShapeDtypeStruct((seq_len, dim), weight.dtype),
        grid=(grid,),
        in_specs=[pl.BlockSpec((_BLOCK_ROWS, dim), lambda i: (i, 0))],
        out_specs=pl.BlockSpec((_BLOCK_ROWS, dim), lambda i: (i, 0)),
    )(weight)


# speedup vs baseline: 44.7258x; 1.0843x over previous
import jax
import jax.numpy as jnp
from jax.experimental import pallas as pl
from jax.experimental.pallas import tpu as pltpu


_BLOCK_ROWS = 1024


def _copy_body(src_ref, dst_ref):
    dst_ref[...] = src_ref[...]


def kernel(inputs, weight):
    bsz, seq_len = inputs.shape[:2]
    dim = weight.shape[1]
    grid = seq_len // _BLOCK_ROWS
    return pl.pallas_call(
        _copy_body,
        out_shape=jax.ShapeDtypeStruct((seq_len, dim), weight.dtype),
        grid=(grid,),
        in_specs=[pl.BlockSpec((_BLOCK_ROWS, dim), lambda i: (i, 0))],
        out_specs=pl.BlockSpec((_BLOCK_ROWS, dim), lambda i: (i, 0)),
    )(weight)


# speedup vs baseline: 47.8160x; 1.0691x over previous
import jax
import jax.numpy as jnp
from jax.experimental import pallas as pl
from jax.experimental.pallas import tpu as pltpu


_BLOCK_ROWS = 2048


def _copy_body(src_ref, dst_ref):
    dst_ref[...] = src_ref[...]


def kernel(inputs, weight):
    bsz, seq_len = inputs.shape[:2]
    dim = weight.shape[1]
    grid = seq_len // _BLOCK_ROWS
    return pl.pallas_call(
        _copy_body,
        out_shape=jax.ShapeDtypeStruct((seq_len, dim), weight.dtype),
        grid=(grid,),
        in_specs=[pl.BlockSpec((_BLOCK_ROWS, dim), lambda i: (i, 0))],
        out_specs=pl.BlockSpec((_BLOCK_ROWS, dim), lambda i: (i, 0)),
    )(weight)
